# Initial kernel scaffold; baseline (speedup 1.0000x reference)
import jax
import jax.numpy as jnp
from jax.experimental import pallas as pl


def kernel(x, pos, batch, W1a, b1a, W1b, b1b, W2a, b2a, W2b, b2b, W3a, b3a, W3b, b3b, Wr, br):
    raise NotImplementedError("write your pallas kernel here")


# speedup vs baseline: 15.3194x; 15.3194x over previous
import functools
import jax
import jax.numpy as jnp
from jax import lax
from jax.experimental import pallas as pl
from jax.experimental.pallas import tpu as pltpu
from jax.experimental.pallas import tpu_sc as plsc

N = 10000
K = 6
NG = 16
CH = 32
R = 200
NBLK = N // R
CT = 512
NT = 20
NP = NT * CT
NW = 32
BPW = NP // NW
IPC = 128
NCHUNK = BPW * K // IPC
BIGI = 2**30


def _knn_body(lims_ref, pe_ref, brow_ref, post_ref, bcol_ref, nbr_ref, d2_ref):
    r = pl.program_id(0)
    lo = lims_ref[r, 0]
    hi = lims_ref[r, 1]

    brow = brow_ref[...]
    px = pe_ref[:, 0:1]
    py = pe_ref[:, 1:2]
    pz = pe_ref[:, 2:3]

    def dist_tile(t, c):
        tile = post_ref[t]
        qx = tile[0:1, :]
        qy = tile[1:2, :]
        qz = tile[2:3, :]
        d2 = (px - qx) ** 2 + (py - qy) ** 2 + (pz - qz) ** 2
        bc = bcol_ref[t]
        d2 = jnp.where(brow != bc, jnp.inf, d2)
        d2_ref[t] = d2
        return c

    lax.fori_loop(lo, hi, dist_tile, 0)

    lane = lax.broadcasted_iota(jnp.int32, (R, CT), 1)
    prev = jnp.full((R, 1), -1, jnp.int32)
    for p in range(K):
        def pick_tile(t, carry, prev=prev, p=p):
            bd, bi = carry
            d2 = d2_ref[t]
            colidx = lane + t * CT
            if p > 0:
                d2 = jnp.where(colidx == prev, jnp.inf, d2)
                d2_ref[t] = d2
            tmin = jnp.min(d2, axis=1, keepdims=True)
            targ = jnp.min(jnp.where(d2 == tmin, colidx, BIGI),
                           axis=1, keepdims=True)
            upd = tmin < bd
            return (jnp.where(upd, tmin, bd), jnp.where(upd, targ, bi))

        init = (jnp.full((R, 1), jnp.inf, jnp.float32),
                jnp.full((R, 1), -1, jnp.int32))
        _, bi = lax.fori_loop(lo, hi, pick_tile, init)
        nbr_ref[:, p:p + 1] = bi
        prev = bi


def _knn(pos, batch, seg):
    rstart = jnp.arange(NBLK, dtype=jnp.int32) * R
    g0 = batch[rstart]
    g1 = batch[rstart + R - 1]
    lo = seg[g0] // CT
    hi = (seg[g1 + 1] + CT - 1) // CT
    lims = jnp.stack([lo, hi], axis=1)

    post = jnp.pad(pos, ((0, NP - N), (0, 0))).T.reshape(3, NT, CT)
    post = post.transpose(1, 0, 2)
    bcol = jnp.pad(batch, (0, NP - N),
                   constant_values=-1).reshape(NT, 1, CT)
    brow = batch.reshape(N, 1)

    return pl.pallas_call(
        _knn_body,
        grid=(NBLK,),
        in_specs=[
            pl.BlockSpec(memory_space=pltpu.SMEM),
            pl.BlockSpec((R, 3), lambda r: (r, 0)),
            pl.BlockSpec((R, 1), lambda r: (r, 0)),
            pl.BlockSpec((NT, 3, CT), lambda r: (0, 0, 0)),
            pl.BlockSpec((NT, 1, CT), lambda r: (0, 0, 0)),
        ],
        out_specs=pl.BlockSpec((R, K), lambda r: (r, 0)),
        out_shape=jax.ShapeDtypeStruct((N, K), jnp.int32),
        scratch_shapes=[pltpu.VMEM((NT, R, CT), jnp.float32)],
    )(lims, pos, brow, post, bcol)


def _mlp_body(h_ref, wa_ref, ba_ref, wb_ref, bb_ref, out_ref):
    t = jnp.dot(h_ref[...], wa_ref[...],
                preferred_element_type=jnp.float32) + ba_ref[...]
    t = jnp.maximum(t, 0.0)
    out_ref[...] = jnp.dot(t, wb_ref[...],
                           preferred_element_type=jnp.float32) + bb_ref[...]


def _mlp(h, wa, ba, wb, bb):
    return pl.pallas_call(
        _mlp_body,
        out_shape=jax.ShapeDtypeStruct((h.shape[0], CH), jnp.float32),
    )(h, wa, ba.reshape(1, -1), wb, bb.reshape(1, -1))


def _sc_gather_body(m_hbm, idx_hbm, out_hbm, idx_v, rows_v, out_v, sem):
    wid = lax.axis_index("s") * 2 + lax.axis_index("c")
    pltpu.sync_copy(idx_hbm.at[wid], idx_v)
    copies = []
    for c in range(NCHUNK):
        copies.append(pltpu.async_copy(
            m_hbm.at[idx_v.at[c]],
            rows_v.at[pl.ds(c * IPC, IPC)], sem))
    for cp in copies:
        cp.wait()

    def node(n, carry):
        base = n * K
        for c2 in range(CH // 16):
            sl = pl.ds(c2 * 16, 16)
            acc = rows_v[base, sl]
            for j in range(1, K):
                acc = jnp.maximum(acc, rows_v[base + j, sl])
            out_v[n, sl] = jnp.maximum(acc, 0.0)
        return carry

    lax.fori_loop(0, BPW, node, 0)
    pltpu.sync_copy(out_v, out_hbm.at[pl.ds(wid * BPW, BPW)])


@functools.lru_cache(maxsize=None)
def _sc_gather_kernel():
    return functools.partial(
        pl.kernel,
        mesh=plsc.VectorSubcoreMesh(core_axis_name="c", subcore_axis_name="s"),
        out_type=jax.ShapeDtypeStruct((NP, CH), jnp.float32),
        scratch_types=[
            pltpu.VMEM((NCHUNK, IPC), jnp.int32),
            pltpu.VMEM((BPW * K, CH), jnp.float32),
            pltpu.VMEM((BPW, CH), jnp.float32),
            pltpu.SemaphoreType.DMA,
        ],
        compiler_params=pltpu.CompilerParams(use_tc_tiling_on_sc=False),
    )(_sc_gather_body)


def _sc_gather_max(m, nbrf):
    return _sc_gather_kernel()(m, nbrf)


PT = 256
NPT = NP // PT


def _pool_body(seg_ref, h_ref, bcol_ref, wr_ref, br_ref, out_ref, acc_ref):
    def per_graph(g, c):
        t0 = seg_ref[g] // PT
        t1 = (seg_ref[g + 1] + PT - 1) // PT

        def per_tile(t, acc):
            tile = h_ref[t]
            bt = bcol_ref[t]
            m = jnp.where(bt == g, tile, -jnp.inf)
            return jnp.maximum(acc, jnp.max(m, axis=0, keepdims=True))

        acc = lax.fori_loop(t0, t1, per_tile,
                            jnp.full((1, CH), -jnp.inf, jnp.float32))
        acc_ref[pl.ds(g, 1), :] = acc
        return c

    lax.fori_loop(0, NG, per_graph, 0)
    out_ref[...] = jnp.dot(acc_ref[...], wr_ref[...],
                           preferred_element_type=jnp.float32) + br_ref[...]


def _pool(seg, h, batch_pad, wr, br):
    return pl.pallas_call(
        _pool_body,
        in_specs=[
            pl.BlockSpec(memory_space=pltpu.SMEM),
            pl.BlockSpec((NPT, PT, CH), lambda: (0, 0, 0)),
            pl.BlockSpec((NPT, PT, 1), lambda: (0, 0, 0)),
            pl.BlockSpec((CH, wr.shape[1]), lambda: (0, 0)),
            pl.BlockSpec((1, wr.shape[1]), lambda: (0, 0)),
        ],
        out_shape=jax.ShapeDtypeStruct((NG, wr.shape[1]), jnp.float32),
        scratch_shapes=[pltpu.VMEM((NG, CH), jnp.float32)],
    )(seg, h.reshape(NPT, PT, CH), batch_pad.reshape(NPT, PT, 1),
      wr, br.reshape(1, -1))


def kernel(x, pos, batch, W1a, b1a, W1b, b1b, W2a, b2a, W2b, b2b,
           W3a, b3a, W3b, b3b, Wr, br):
    batch = batch.astype(jnp.int32)
    seg = jnp.searchsorted(batch, jnp.arange(NG + 1, dtype=jnp.int32),
                           side='left').astype(jnp.int32)
    nbr = _knn(pos, batch, seg)
    nbrf = jnp.pad(nbr, ((0, NP - N), (0, 0))).reshape(NW, NCHUNK, IPC)

    h = jnp.pad(jnp.concatenate([x, pos], axis=1), ((0, NP - N), (0, 0)))
    h = _sc_gather_max(_mlp(h, W1a, b1a, W1b, b1b), nbrf)
    h = _sc_gather_max(_mlp(h, W2a, b2a, W2b, b2b), nbrf)
    h = _sc_gather_max(_mlp(h, W3a, b3a, W3b, b3b), nbrf)

    batch_pad = jnp.pad(batch, (0, NP - N), constant_values=-1)
    return _pool(seg, h, batch_pad, Wr, br)
